# baseline (speedup 1.0000x reference)
import jax
import jax.numpy as jnp
import numpy as np
from jax import lax
from jax.experimental import pallas as pl
from jax.experimental.pallas import tpu as pltpu
from jax.experimental.pallas import tpu_sc as plsc

N = 10000
E = 320000
DF = 128
DE = 16
H = 64
HW2 = H // 2
L = 3
G = 64

NC = 2
NS = 16
NW = NC * NS
CHUNK = 128
EPW = 10240
EP = NW * EPW
NCHUNKS = EPW // CHUNK
NP = 10112
RPT = NP // NS
SPT = N // NS
_ROW_CHUNKS = ((0, 128), (128, 128), (256, 128), (384, 128), (512, 120))

_QL = np.concatenate([np.arange(0, 16), np.arange(32, 48)])
_QH = np.concatenate([np.arange(16, 32), np.arange(48, 64)])
_QLH = np.concatenate([_QL, _QH]).astype(np.int32)


SUP = 2
SE = SUP * CHUNK
NSTEPS = EPW // SE


def _sc_body(hb_hbm, src2_hbm, dst2_hbm, embb_hbm, out_hbm,
             src_all, dst_all, dumidx, hrow0, hrow1, embb0, embb1,
             rows0, rows1, agg_s,
             semg0, semg1, seme0, seme1, sems):
    c = lax.axis_index("c")
    s = lax.axis_index("s")
    w = s * NC + c
    hrow_ = (hrow0, hrow1)
    embb_ = (embb0, embb1)
    rows_ = (rows0, rows1)
    semg_ = (semg0, semg1)
    seme_ = (seme0, seme1)

    pltpu.sync_copy(src2_hbm.at[pl.ds(w * NCHUNKS, NCHUNKS)], src_all)
    pltpu.sync_copy(dst2_hbm.at[pl.ds(w * NCHUNKS, NCHUNKS)], dst_all)

    def _zrow(i, carry):
        for k in range(4):
            rows0[i, pl.ds(16 * k, 16)] = jnp.zeros((16,), jnp.float32)
        return carry
    lax.fori_loop(0, CHUNK, _zrow, 0, unroll=4)
    base_r = s * RPT
    for off, nr in _ROW_CHUNKS:
        pltpu.sync_copy(rows0.at[pl.ds(0, nr)], agg_s.at[pl.ds(base_r + off, nr)])
    for k in range(8):
        dumidx[0, pl.ds(16 * k, 16)] = jnp.full((16,), N, jnp.int32)
    plsc.subcore_barrier()

    def _start_fetch(b, i):
        for s_ in range(SUP):
            for hh in range(2):
                pltpu.async_copy(
                    hb_hbm.at[src_all.at[SUP * i + s_, pl.ds(64 * hh, 64)]],
                    hrow_[b].at[pl.ds(CHUNK * s_ + 64 * hh, 64)], semg_[b])
        e0 = (w * NCHUNKS + SUP * i) * CHUNK
        pltpu.async_copy(embb_hbm.at[pl.ds(e0, SE)], embb_[b], seme_[b])

    def _wait_fetch(b):
        for s_ in range(SUP):
            for hh in range(2):
                pltpu.make_async_copy(
                    hb_hbm.at[src_all.at[s_, pl.ds(64 * hh, 64)]],
                    hrow_[b].at[pl.ds(CHUNK * s_ + 64 * hh, 64)],
                    semg_[b]).wait()
        pltpu.make_async_copy(embb_hbm.at[pl.ds(0, SE)], embb_[b],
                              seme_[b]).wait()

    def _wait_scatter_pair():
        for s_ in range(SUP):
            pltpu.make_async_copy(rows0.at[pl.ds(CHUNK * s_, CHUNK)],
                                  agg_s.at[dumidx.at[0]], sems).wait()

    for s_ in range(SUP):
        pltpu.async_copy(rows0.at[pl.ds(CHUNK * s_, CHUNK)],
                         agg_s.at[dumidx.at[0]], sems, add=True)
    _start_fetch(0, jnp.int32(0))

    def _outer(g, carry):
        for b in range(2):
            i = 2 * g + b
            _wait_fetch(b)
            _wait_scatter_pair()
            inext = jnp.minimum(i + 1, NSTEPS - 1)
            _start_fetch(1 - b, inext)

            def _vrow(j, cy):
                for gg in range(H // 32):
                    hsl = pl.ds(16 * gg, 16)
                    hw = plsc.bitcast(hrow_[b][j, hsl], jnp.bfloat16)
                    ew = plsc.bitcast(embb_[b][j, hsl], jnp.bfloat16)
                    sm = jnp.maximum(hw + ew, jnp.bfloat16(0.0))
                    pa, pb = plsc.unpack(sm, format=plsc.PackFormat.INTERLEAVED)
                    rows_[b][j, pl.ds(32 * gg, 16)] = pa
                    rows_[b][j, pl.ds(32 * gg + 16, 16)] = pb
                return cy
            lax.fori_loop(0, SE, _vrow, 0, unroll=2)
            for s_ in range(SUP):
                pltpu.async_copy(rows_[b].at[pl.ds(CHUNK * s_, CHUNK)],
                                 agg_s.at[dst_all.at[SUP * i + s_]], sems, add=True)
        return carry
    lax.fori_loop(0, NSTEPS // 2, _outer, 0)
    _wait_fetch(0)
    _wait_scatter_pair()
    plsc.subcore_barrier()

    for off, nr in _ROW_CHUNKS:
        pltpu.sync_copy(agg_s.at[pl.ds(base_r + off, nr)], rows0.at[pl.ds(0, nr)])
        pltpu.sync_copy(rows0.at[pl.ds(0, nr)], out_hbm.at[c, pl.ds(base_r + off, nr)])


def _make_sc_layer():
    mesh = plsc.VectorSubcoreMesh(core_axis_name="c", subcore_axis_name="s")
    return pl.kernel(
        _sc_body,
        out_type=jax.ShapeDtypeStruct((NC, NP, H), jnp.float32),
        mesh=mesh,
        scratch_types=[
            pltpu.VMEM((NCHUNKS, CHUNK), jnp.int32),
            pltpu.VMEM((NCHUNKS, CHUNK), jnp.int32),
            pltpu.VMEM((1, CHUNK), jnp.int32),
            pltpu.VMEM((SE, HW2), jnp.int32),
            pltpu.VMEM((SE, HW2), jnp.int32),
            pltpu.VMEM((SE, HW2), jnp.int32),
            pltpu.VMEM((SE, HW2), jnp.int32),
            pltpu.VMEM((SE, H), jnp.float32),
            pltpu.VMEM((SE, H), jnp.float32),
            pltpu.VMEM_SHARED((NP, H), jnp.float32),
            pltpu.SemaphoreType.DMA,
            pltpu.SemaphoreType.DMA,
            pltpu.SemaphoreType.DMA,
            pltpu.SemaphoreType.DMA,
            pltpu.SemaphoreType.DMA,
        ],
        compiler_params=pltpu.CompilerParams(use_tc_tiling_on_sc=False,
                                             needs_layout_passes=False),
    )


_SC_LAYER = _make_sc_layer()


def _pack_i32(hq):
    lo = lax.bitcast_convert_type(hq[..., :32].astype(jnp.bfloat16),
                                  jnp.uint16).astype(jnp.uint32)
    hi = lax.bitcast_convert_type(hq[..., 32:].astype(jnp.bfloat16),
                                  jnp.uint16).astype(jnp.uint32)
    return lax.bitcast_convert_type(lo | (hi << 16), jnp.int32)


def _enc_kernel(x_ref, w_ref, b_ref, wq_ref, bq_ref, o_ref, ob_ref):
    x = x_ref[...]
    o_ref[...] = jnp.dot(x, w_ref[...],
                         preferred_element_type=jnp.float32) + b_ref[...]
    hbq = _pack_i32(jnp.dot(x, wq_ref[...],
                            preferred_element_type=jnp.float32) + bq_ref[...])
    ob_ref[:N] = hbq
    ob_ref[N:] = jnp.zeros((NP - N, HW2), jnp.int32)


def _eemb_kernel(a_ref, wq_ref, bq_ref, ob_ref):
    ob_ref[...] = _pack_i32(jnp.dot(a_ref[...], wq_ref[0],
                                    preferred_element_type=jnp.float32)
                            + bq_ref[0])


def _bn(a, g, b):
    m = jnp.mean(a, axis=0)
    v = jnp.mean((a - m) ** 2, axis=0)
    return (a - m) / jnp.sqrt(v + 1e-5) * g + b


def _mlp_kernel(h_ref, aggp_ref, eps_ref, w1_ref, b1_ref, g1_ref, bb1_ref,
                w2_ref, b2_ref, g2_ref, bb2_ref,
                w2q_ref, b2q_ref, g2q_ref, bb2q_ref, o_ref, ob_ref):
    h = h_ref[...]
    agg = aggp_ref[0, :N, :] + aggp_ref[1, :N, :]
    z = (1.0 + eps_ref[0]) * h + agg
    a = jnp.dot(z, w1_ref[...], preferred_element_type=jnp.float32) + b1_ref[...]
    a = jnp.maximum(_bn(a, g1_ref[...], bb1_ref[...]), 0.0)
    cz = jnp.dot(a, w2_ref[...], preferred_element_type=jnp.float32) + b2_ref[...]
    o_ref[...] = jnp.maximum(_bn(cz, g2_ref[...], bb2_ref[...]), 0.0)
    czq = jnp.dot(a, w2q_ref[...], preferred_element_type=jnp.float32) + b2q_ref[...]
    ob_ref[:N] = _pack_i32(jnp.maximum(_bn(czq, g2q_ref[...], bb2q_ref[...]), 0.0))
    ob_ref[N:] = jnp.zeros((NP - N, HW2), jnp.int32)


def _pool_kernel(h_ref, batch_ref, o_ref):
    b2 = batch_ref[...]
    gids = lax.broadcasted_iota(jnp.int32, (N, G), 1)
    oh = (b2 == gids).astype(jnp.float32)
    sums = lax.dot_general(oh, h_ref[...], (((0,), (0,)), ((), ())),
                           preferred_element_type=jnp.float32)
    counts = jnp.sum(oh, axis=0)
    o_ref[...] = sums / jnp.maximum(counts, 1.0)[:, None]


def kernel(x, edge_index, edge_attr, batch, enc_W, enc_b, edge_W, edge_b, eps,
           mlp1_W, mlp1_b, mlpbn_g, mlpbn_b, mlp2_W, mlp2_b, bn_g, bn_b):
    src = edge_index[0]
    dst = edge_index[1]
    pad = EP - E
    src2 = jnp.concatenate([src, jnp.zeros((pad,), jnp.int32)]).reshape(EP // CHUNK, CHUNK)
    dst2 = jnp.concatenate([dst, jnp.full((pad,), N, jnp.int32)]).reshape(EP // CHUNK, CHUNK)
    attr_p = jnp.concatenate([edge_attr, jnp.zeros((pad, DE), jnp.float32)])
    q = jnp.asarray(_QLH)

    h, hb = pl.pallas_call(
        _enc_kernel,
        out_shape=(jax.ShapeDtypeStruct((N, H), jnp.float32),
                   jax.ShapeDtypeStruct((NP, HW2), jnp.int32)),
    )(x, enc_W, enc_b, enc_W[:, q], enc_b[q])

    BE = 4096
    eemb_call = pl.pallas_call(
        _eemb_kernel,
        grid=(EP // BE,),
        in_specs=[
            pl.BlockSpec((BE, DE), lambda b: (b, 0)),
            pl.BlockSpec((1, DE, H), lambda b: (0, 0, 0)),
            pl.BlockSpec((1, 1, H), lambda b: (0, 0, 0)),
        ],
        out_specs=pl.BlockSpec((BE, HW2), lambda b: (b, 0)),
        out_shape=jax.ShapeDtypeStruct((EP, HW2), jnp.int32),
    )

    mlp_call = pl.pallas_call(
        _mlp_kernel,
        in_specs=[pl.BlockSpec(memory_space=pltpu.VMEM)] * 2
        + [pl.BlockSpec(memory_space=pltpu.SMEM)]
        + [pl.BlockSpec(memory_space=pltpu.VMEM)] * 12,
        out_shape=(jax.ShapeDtypeStruct((N, H), jnp.float32),
                   jax.ShapeDtypeStruct((NP, HW2), jnp.int32)),
    )

    def _layer(carry, xs):
        h, hb = carry
        (eWq, ebq, ep_, w1, b1, g1, bb1, w2, b2, g2, bb2,
         w2q, b2q, g2q, bb2q) = xs
        embb_l = eemb_call(attr_p, eWq.reshape(1, DE, H), ebq.reshape(1, 1, H))
        aggp = _SC_LAYER(hb, src2, dst2, embb_l)
        h, hb = mlp_call(h, aggp, ep_.reshape(1), w1, b1, g1, bb1,
                         w2, b2, g2, bb2, w2q, b2q, g2q, bb2q)
        return (h, hb), None

    (h, _), _ = lax.scan(
        _layer, (h, hb),
        (edge_W[:, :, q], edge_b[:, q], eps, mlp1_W, mlp1_b,
         mlpbn_g, mlpbn_b, mlp2_W, mlp2_b, bn_g, bn_b,
         mlp2_W[:, :, q], mlp2_b[:, q], bn_g[:, q], bn_b[:, q]))

    return pl.pallas_call(
        _pool_kernel,
        out_shape=jax.ShapeDtypeStruct((G, H), jnp.float32),
    )(h, batch.reshape(N, 1))


# speedup vs baseline: 1.0007x; 1.0007x over previous
import jax
import jax.numpy as jnp
import numpy as np
from jax import lax
from jax.experimental import pallas as pl
from jax.experimental.pallas import tpu as pltpu
from jax.experimental.pallas import tpu_sc as plsc

N = 10000
E = 320000
DF = 128
DE = 16
H = 64
HW2 = H // 2
L = 3
G = 64

NC = 2
NS = 16
NW = NC * NS
CHUNK = 128
EPW = 10240
EP = NW * EPW
NCHUNKS = EPW // CHUNK
NP = 10112
RPT = NP // NS
_ROW_CHUNKS = ((0, 128), (128, 128), (256, 128), (384, 128), (512, 120))

_QL = np.concatenate([np.arange(0, 16), np.arange(32, 48)])
_QH = np.concatenate([np.arange(16, 32), np.arange(48, 64)])
_QLH = np.concatenate([_QL, _QH]).astype(np.int32)


SUP = 2
SE = SUP * CHUNK
NSTEPS = EPW // SE


def _sc_body(hb_hbm, src2_hbm, dst2_hbm, embb_hbm, out_hbm,
             src_all, dst_all, dumidx, hrow0, hrow1, embb0, embb1,
             rows0, rows1, agg_s,
             semg0, semg1, seme0, seme1, sems):
    c = lax.axis_index("c")
    s = lax.axis_index("s")
    w = s * NC + c
    hrow_ = (hrow0, hrow1)
    embb_ = (embb0, embb1)
    rows_ = (rows0, rows1)
    semg_ = (semg0, semg1)
    seme_ = (seme0, seme1)

    pltpu.sync_copy(src2_hbm.at[pl.ds(w * NCHUNKS, NCHUNKS)], src_all)
    pltpu.sync_copy(dst2_hbm.at[pl.ds(w * NCHUNKS, NCHUNKS)], dst_all)

    def _zrow(i, carry):
        for k in range(4):
            rows0[i, pl.ds(16 * k, 16)] = jnp.zeros((16,), jnp.float32)
        return carry
    lax.fori_loop(0, CHUNK, _zrow, 0, unroll=4)
    base_r = s * RPT
    for off, nr in _ROW_CHUNKS:
        pltpu.sync_copy(rows0.at[pl.ds(0, nr)], agg_s.at[pl.ds(base_r + off, nr)])
    for k in range(8):
        dumidx[0, pl.ds(16 * k, 16)] = jnp.full((16,), N, jnp.int32)
    plsc.subcore_barrier()

    def _start_fetch(b, i):
        for s_ in range(SUP):
            pltpu.async_copy(hb_hbm.at[src_all.at[SUP * i + s_]],
                             hrow_[b].at[pl.ds(CHUNK * s_, CHUNK)], semg_[b])
        e0 = (w * NCHUNKS + SUP * i) * CHUNK
        pltpu.async_copy(embb_hbm.at[pl.ds(e0, SE)], embb_[b], seme_[b])

    def _wait_fetch(b):
        for s_ in range(SUP):
            pltpu.make_async_copy(hb_hbm.at[src_all.at[s_]],
                                  hrow_[b].at[pl.ds(CHUNK * s_, CHUNK)],
                                  semg_[b]).wait()
        pltpu.make_async_copy(embb_hbm.at[pl.ds(0, SE)], embb_[b],
                              seme_[b]).wait()

    def _wait_scatter_pair():
        for s_ in range(SUP):
            pltpu.make_async_copy(rows0.at[pl.ds(CHUNK * s_, CHUNK)],
                                  agg_s.at[dumidx.at[0]], sems).wait()

    for s_ in range(SUP):
        pltpu.async_copy(rows0.at[pl.ds(CHUNK * s_, CHUNK)],
                         agg_s.at[dumidx.at[0]], sems, add=True)
    _start_fetch(0, jnp.int32(0))

    def _outer(g, carry):
        for b in range(2):
            i = 2 * g + b
            _wait_fetch(b)
            _wait_scatter_pair()
            inext = jnp.minimum(i + 1, NSTEPS - 1)
            _start_fetch(1 - b, inext)

            def _vrow(j, cy):
                for gg in range(H // 32):
                    hsl = pl.ds(16 * gg, 16)
                    hw = plsc.bitcast(hrow_[b][j, hsl], jnp.bfloat16)
                    ew = plsc.bitcast(embb_[b][j, hsl], jnp.bfloat16)
                    sm = jnp.maximum(hw + ew, jnp.bfloat16(0.0))
                    pa, pb = plsc.unpack(sm, format=plsc.PackFormat.INTERLEAVED)
                    rows_[b][j, pl.ds(32 * gg, 16)] = pa
                    rows_[b][j, pl.ds(32 * gg + 16, 16)] = pb
                return cy
            lax.fori_loop(0, SE, _vrow, 0, unroll=2)
            for s_ in range(SUP):
                pltpu.async_copy(rows_[b].at[pl.ds(CHUNK * s_, CHUNK)],
                                 agg_s.at[dst_all.at[SUP * i + s_]], sems, add=True)
        return carry
    lax.fori_loop(0, NSTEPS // 2, _outer, 0)
    _wait_fetch(0)
    _wait_scatter_pair()
    plsc.subcore_barrier()

    for off, nr in _ROW_CHUNKS:
        pltpu.sync_copy(agg_s.at[pl.ds(base_r + off, nr)], rows0.at[pl.ds(0, nr)])
        pltpu.sync_copy(rows0.at[pl.ds(0, nr)], out_hbm.at[c, pl.ds(base_r + off, nr)])


def _make_sc_layer():
    mesh = plsc.VectorSubcoreMesh(core_axis_name="c", subcore_axis_name="s")
    return pl.kernel(
        _sc_body,
        out_type=jax.ShapeDtypeStruct((NC, NP, H), jnp.float32),
        mesh=mesh,
        scratch_types=[
            pltpu.VMEM((NCHUNKS, CHUNK), jnp.int32),
            pltpu.VMEM((NCHUNKS, CHUNK), jnp.int32),
            pltpu.VMEM((1, CHUNK), jnp.int32),
            pltpu.VMEM((SE, HW2), jnp.int32),
            pltpu.VMEM((SE, HW2), jnp.int32),
            pltpu.VMEM((SE, HW2), jnp.int32),
            pltpu.VMEM((SE, HW2), jnp.int32),
            pltpu.VMEM((SE, H), jnp.float32),
            pltpu.VMEM((SE, H), jnp.float32),
            pltpu.VMEM_SHARED((NP, H), jnp.float32),
            pltpu.SemaphoreType.DMA,
            pltpu.SemaphoreType.DMA,
            pltpu.SemaphoreType.DMA,
            pltpu.SemaphoreType.DMA,
            pltpu.SemaphoreType.DMA,
        ],
        compiler_params=pltpu.CompilerParams(use_tc_tiling_on_sc=False,
                                             needs_layout_passes=False),
    )


_SC_LAYER = _make_sc_layer()


def _pack_i32(hq):
    lo = lax.bitcast_convert_type(hq[..., :32].astype(jnp.bfloat16),
                                  jnp.uint16).astype(jnp.uint32)
    hi = lax.bitcast_convert_type(hq[..., 32:].astype(jnp.bfloat16),
                                  jnp.uint16).astype(jnp.uint32)
    return lax.bitcast_convert_type(lo | (hi << 16), jnp.int32)


def _enc_kernel(x_ref, w_ref, b_ref, wq_ref, bq_ref, o_ref, ob_ref):
    x = x_ref[...]
    o_ref[...] = jnp.dot(x, w_ref[...],
                         preferred_element_type=jnp.float32) + b_ref[...]
    hbq = _pack_i32(jnp.dot(x, wq_ref[...],
                            preferred_element_type=jnp.float32) + bq_ref[...])
    ob_ref[:N] = hbq
    ob_ref[N:] = jnp.zeros((NP - N, HW2), jnp.int32)


def _eemb_kernel(a_ref, wq_ref, bq_ref, ob_ref):
    ob_ref[...] = _pack_i32(jnp.dot(a_ref[...], wq_ref[0],
                                    preferred_element_type=jnp.float32)
                            + bq_ref[0])


def _bn(a, g, b):
    m = jnp.mean(a, axis=0)
    v = jnp.mean((a - m) ** 2, axis=0)
    return (a - m) / jnp.sqrt(v + 1e-5) * g + b


def _mlp_kernel(h_ref, aggp_ref, eps_ref, w1_ref, b1_ref, g1_ref, bb1_ref,
                w2_ref, b2_ref, g2_ref, bb2_ref,
                w2q_ref, b2q_ref, g2q_ref, bb2q_ref, o_ref, ob_ref):
    h = h_ref[...]
    agg = aggp_ref[0, :N, :] + aggp_ref[1, :N, :]
    z = (1.0 + eps_ref[0]) * h + agg
    a = jnp.dot(z, w1_ref[...], preferred_element_type=jnp.float32) + b1_ref[...]
    a = jnp.maximum(_bn(a, g1_ref[...], bb1_ref[...]), 0.0)
    cz = jnp.dot(a, w2_ref[...], preferred_element_type=jnp.float32) + b2_ref[...]
    o_ref[...] = jnp.maximum(_bn(cz, g2_ref[...], bb2_ref[...]), 0.0)
    czq = jnp.dot(a, w2q_ref[...], preferred_element_type=jnp.float32) + b2q_ref[...]
    ob_ref[:N] = _pack_i32(jnp.maximum(_bn(czq, g2q_ref[...], bb2q_ref[...]), 0.0))
    ob_ref[N:] = jnp.zeros((NP - N, HW2), jnp.int32)


def _pool_kernel(h_ref, batch_ref, o_ref):
    b2 = batch_ref[...]
    gids = lax.broadcasted_iota(jnp.int32, (N, G), 1)
    oh = (b2 == gids).astype(jnp.float32)
    sums = lax.dot_general(oh, h_ref[...], (((0,), (0,)), ((), ())),
                           preferred_element_type=jnp.float32)
    counts = jnp.sum(oh, axis=0)
    o_ref[...] = sums / jnp.maximum(counts, 1.0)[:, None]


def kernel(x, edge_index, edge_attr, batch, enc_W, enc_b, edge_W, edge_b, eps,
           mlp1_W, mlp1_b, mlpbn_g, mlpbn_b, mlp2_W, mlp2_b, bn_g, bn_b):
    src = edge_index[0]
    dst = edge_index[1]
    pad = EP - E
    src2 = jnp.concatenate([src, jnp.zeros((pad,), jnp.int32)]).reshape(EP // CHUNK, CHUNK)
    dst2 = jnp.concatenate([dst, jnp.full((pad,), N, jnp.int32)]).reshape(EP // CHUNK, CHUNK)
    attr_p = jnp.concatenate([edge_attr, jnp.zeros((pad, DE), jnp.float32)])
    q = jnp.asarray(_QLH)

    h, hb = pl.pallas_call(
        _enc_kernel,
        out_shape=(jax.ShapeDtypeStruct((N, H), jnp.float32),
                   jax.ShapeDtypeStruct((NP, HW2), jnp.int32)),
    )(x, enc_W, enc_b, enc_W[:, q], enc_b[q])

    BE = 4096
    eemb_call = pl.pallas_call(
        _eemb_kernel,
        grid=(EP // BE,),
        in_specs=[
            pl.BlockSpec((BE, DE), lambda b: (b, 0)),
            pl.BlockSpec((1, DE, H), lambda b: (0, 0, 0)),
            pl.BlockSpec((1, 1, H), lambda b: (0, 0, 0)),
        ],
        out_specs=pl.BlockSpec((BE, HW2), lambda b: (b, 0)),
        out_shape=jax.ShapeDtypeStruct((EP, HW2), jnp.int32),
    )

    mlp_call = pl.pallas_call(
        _mlp_kernel,
        in_specs=[pl.BlockSpec(memory_space=pltpu.VMEM)] * 2
        + [pl.BlockSpec(memory_space=pltpu.SMEM)]
        + [pl.BlockSpec(memory_space=pltpu.VMEM)] * 12,
        out_shape=(jax.ShapeDtypeStruct((N, H), jnp.float32),
                   jax.ShapeDtypeStruct((NP, HW2), jnp.int32)),
    )

    def _layer(carry, xs):
        h, hb = carry
        (eWq, ebq, ep_, w1, b1, g1, bb1, w2, b2, g2, bb2,
         w2q, b2q, g2q, bb2q) = xs
        embb_l = eemb_call(attr_p, eWq.reshape(1, DE, H), ebq.reshape(1, 1, H))
        aggp = _SC_LAYER(hb, src2, dst2, embb_l)
        h, hb = mlp_call(h, aggp, ep_.reshape(1), w1, b1, g1, bb1,
                         w2, b2, g2, bb2, w2q, b2q, g2q, bb2q)
        return (h, hb), None

    (h, _), _ = lax.scan(
        _layer, (h, hb),
        (edge_W[:, :, q], edge_b[:, q], eps, mlp1_W, mlp1_b,
         mlpbn_g, mlpbn_b, mlp2_W, mlp2_b, bn_g, bn_b,
         mlp2_W[:, :, q], mlp2_b[:, q], bn_g[:, q], bn_b[:, q]))

    return pl.pallas_call(
        _pool_kernel,
        out_shape=jax.ShapeDtypeStruct((G, H), jnp.float32),
    )(h, batch.reshape(N, 1))
